# Initial kernel scaffold; baseline (speedup 1.0000x reference)
import jax
import jax.numpy as jnp
from jax.experimental import pallas as pl


def kernel(query, reference_points, value, value_spatial_shapes, W_off, b_off, W_attn, b_attn):
    raise NotImplementedError("write your pallas kernel here")


# speedup vs baseline: 9.1561x; 9.1561x over previous
import functools

import jax
import jax.numpy as jnp
from jax import lax
from jax.experimental import pallas as pl
from jax.experimental.pallas import tpu as pltpu
from jax.experimental.pallas import tpu_sc as plsc

EMBED_DIM = 256
NUM_HEADS = 8
HEAD_DIM = 32
NUM_POINTS_LIST = (3, 6, 3)
SUM_POINTS = 12
NCORNER = 48
GH = GW = 40
BS = 16
LEN_Q = 300
QPAD = 304
BH = BS * NUM_HEADS

KBOUNDS = (0, 12, 36, 48)

NUM_WORKERS = 32
PAIRS_PER_WORKER = BH // NUM_WORKERS


def _prep_body(qT_ref, refT_ref, WoffT_ref, boff_ref, WattnT_ref, battn_ref,
               cells_ref, ws_ref):
    qT = qT_ref[0]
    offsT = jnp.dot(WoffT_ref[...], qT, preferred_element_type=jnp.float32)
    offsT = offsT + boff_ref[...]
    logits = jnp.dot(WattnT_ref[...], qT, preferred_element_type=jnp.float32)
    logits = logits + battn_ref[...]

    lg = logits.reshape(NUM_HEADS, SUM_POINTS, QPAD)
    m = jnp.max(lg, axis=1, keepdims=True)
    e = jnp.exp(lg - m)
    attn = e / jnp.sum(e, axis=1, keepdims=True)

    offs = offsT.reshape(NUM_HEADS, SUM_POINTS, 2, QPAD)
    ox = offs[:, :, 0, :]
    oy = offs[:, :, 1, :]

    refxy = refT_ref[0]
    refx = refxy[0:1, :]
    refy = refxy[1:2, :]
    refw = refxy[2:3, :]
    refh = refxy[3:4, :]

    p_iota = lax.broadcasted_iota(jnp.int32, (1, SUM_POINTS, 1), 1)
    nscale = jnp.where((p_iota >= 3) & (p_iota < 9),
                       jnp.float32(1.0 / 12.0), jnp.float32(1.0 / 6.0))

    locx = refx + ox * nscale * refw
    locy = refy + oy * nscale * refh
    x = locx * GW - 0.5
    y = locy * GH - 0.5
    x0 = jnp.floor(x)
    y0 = jnp.floor(y)
    fx = x - x0
    fy = y - y0
    x0i = x0.astype(jnp.int32)
    y0i = y0.astype(jnp.int32)

    qmask = (lax.broadcasted_iota(jnp.int32, (1, 1, QPAD), 2) < LEN_Q
             ).astype(jnp.float32)

    cell_list = []
    w_list = []
    for dx, dy, wgt in ((0, 0, (1 - fx) * (1 - fy)),
                        (0, 1, (1 - fx) * fy),
                        (1, 0, fx * (1 - fy)),
                        (1, 1, fx * fy)):
        ix = x0i + dx
        iy = y0i + dy
        valid = ((ix >= 0) & (ix < GW) & (iy >= 0) & (iy < GH)
                 ).astype(jnp.float32)
        cell = (jnp.clip(iy, 0, GH - 1) * GW + jnp.clip(ix, 0, GW - 1))
        cell_list.append(cell[:, :, None, :])
        w_list.append((wgt * valid * attn * qmask)[:, :, None, :])

    cells = jnp.concatenate(cell_list, axis=2).reshape(NUM_HEADS, NCORNER, QPAD)
    ws = jnp.concatenate(w_list, axis=2).reshape(NUM_HEADS, NCORNER, QPAD)
    cells_ref[0] = cells
    ws_ref[0] = ws


def _run_prep(qT, refT, WoffT, boff, WattnT, battn):
    grid = (BS,)
    return pl.pallas_call(
        _prep_body,
        grid=grid,
        in_specs=[
            pl.BlockSpec((1, EMBED_DIM, QPAD), lambda b: (b, 0, 0)),
            pl.BlockSpec((1, 4, QPAD), lambda b: (b, 0, 0)),
            pl.BlockSpec((SUM_POINTS * 16, EMBED_DIM), lambda b: (0, 0)),
            pl.BlockSpec((SUM_POINTS * 16, 1), lambda b: (0, 0)),
            pl.BlockSpec((SUM_POINTS * 8, EMBED_DIM), lambda b: (0, 0)),
            pl.BlockSpec((SUM_POINTS * 8, 1), lambda b: (0, 0)),
        ],
        out_specs=[
            pl.BlockSpec((1, NUM_HEADS, NCORNER, QPAD), lambda b: (b, 0, 0, 0)),
            pl.BlockSpec((1, NUM_HEADS, NCORNER, QPAD), lambda b: (b, 0, 0, 0)),
        ],
        out_shape=[
            jax.ShapeDtypeStruct((BS, NUM_HEADS, NCORNER, QPAD), jnp.int32),
            jax.ShapeDtypeStruct((BS, NUM_HEADS, NCORNER, QPAD), jnp.float32),
        ],
    )(qT, refT, WoffT, boff, WattnT, battn)


def _sc_body(value_hbm, cells_hbm, ws_hbm, out_hbm, slab_v, cells_v, ws_v, out_v):
    cidx = lax.axis_index("c")
    sidx = lax.axis_index("s")
    wid = sidx * 2 + cidx

    for i in range(PAIRS_PER_WORKER):
        bh = wid * PAIRS_PER_WORKER + i
        b = bh // NUM_HEADS
        h = bh % NUM_HEADS
        pltpu.sync_copy(cells_hbm.at[bh], cells_v)
        pltpu.sync_copy(ws_hbm.at[bh], ws_v)
        for lvl in range(3):
            pltpu.sync_copy(value_hbm.at[lvl, b, h], slab_v)
            k0, k1 = KBOUNDS[lvl], KBOUNDS[lvl + 1]

            def qb_body(qb, carry, k0=k0, k1=k1, lvl=lvl):
                base = qb * 16

                def k_body(k, acc):
                    cellv = cells_v[k, pl.ds(base, 16)]
                    wv = ws_v[k, pl.ds(base, 16)]
                    out = []
                    for ch in range(HEAD_DIM):
                        g = plsc.load_gather(
                            slab_v, [jnp.full((16,), ch, jnp.int32), cellv])
                        out.append(acc[ch] + g * wv)
                    return out

                if lvl == 0:
                    acc0 = [jnp.zeros((16,), jnp.float32)] * HEAD_DIM
                else:
                    acc0 = [out_v[ch, pl.ds(base, 16)] for ch in range(HEAD_DIM)]
                acc = lax.fori_loop(k0, k1, k_body, acc0)
                for ch in range(HEAD_DIM):
                    out_v[ch, pl.ds(base, 16)] = acc[ch]
                return carry

            lax.fori_loop(0, QPAD // 16, qb_body, 0)
        pltpu.sync_copy(out_v, out_hbm.at[bh])


_sc_call_cache = []


def _get_sc_call():
    if not _sc_call_cache:
        mesh = plsc.VectorSubcoreMesh(core_axis_name="c", subcore_axis_name="s")
        _sc_call_cache.append(functools.partial(
            pl.kernel,
            out_type=jax.ShapeDtypeStruct((BH, HEAD_DIM, QPAD), jnp.float32),
            mesh=mesh,
            compiler_params=pltpu.CompilerParams(use_tc_tiling_on_sc=False,
                                                 needs_layout_passes=False),
            scratch_types=[
                pltpu.VMEM((HEAD_DIM, GH * GW), jnp.float32),
                pltpu.VMEM((NCORNER, QPAD), jnp.int32),
                pltpu.VMEM((NCORNER, QPAD), jnp.float32),
                pltpu.VMEM((HEAD_DIM, QPAD), jnp.float32),
            ],
        )(_sc_body))
    return _sc_call_cache[0]


def kernel(query, reference_points, value, value_spatial_shapes,
           W_off, b_off, W_attn, b_attn):
    del value_spatial_shapes
    qT = jnp.pad(jnp.swapaxes(query, 1, 2), ((0, 0), (0, 0), (0, QPAD - LEN_Q)))
    refT = jnp.pad(jnp.swapaxes(reference_points.reshape(BS, LEN_Q, 4), 1, 2),
                   ((0, 0), (0, 0), (0, QPAD - LEN_Q)))
    WoffT = jnp.swapaxes(W_off, 0, 1)
    WattnT = jnp.swapaxes(W_attn, 0, 1)
    boff = b_off.reshape(-1, 1)
    battn = b_attn.reshape(-1, 1)

    cells, ws = _run_prep(qT, refT, WoffT, boff, WattnT, battn)
    cells = cells.reshape(BH, NCORNER, QPAD)
    ws = ws.reshape(BH, NCORNER, QPAD)

    out = _get_sc_call()(value, cells, ws)

    out = out[:, :, :LEN_Q].reshape(BS, NUM_HEADS, HEAD_DIM, LEN_Q)
    return jnp.transpose(out, (0, 3, 1, 2)).reshape(BS, LEN_Q, EMBED_DIM)


# speedup vs baseline: 10.8232x; 1.1821x over previous
import functools

import jax
import jax.numpy as jnp
from jax import lax
from jax.experimental import pallas as pl
from jax.experimental.pallas import tpu as pltpu
from jax.experimental.pallas import tpu_sc as plsc

EMBED_DIM = 256
NUM_HEADS = 8
HEAD_DIM = 32
NUM_POINTS_LIST = (3, 6, 3)
SUM_POINTS = 12
NCORNER = 48
GH = GW = 40
BS = 16
LEN_Q = 300
QPAD = 304
BH = BS * NUM_HEADS

KBOUNDS = (0, 12, 36, 48)

NUM_WORKERS = 32
PAIRS_PER_WORKER = BH // NUM_WORKERS


def _prep_body(qT_ref, refT_ref, WoffT_ref, boff_ref, WattnT_ref, battn_ref,
               cells_ref, ws_ref):
    qT = qT_ref[0]
    offsT = jnp.dot(WoffT_ref[...], qT, preferred_element_type=jnp.float32)
    offsT = offsT + boff_ref[...]
    logits = jnp.dot(WattnT_ref[...], qT, preferred_element_type=jnp.float32)
    logits = logits + battn_ref[...]

    lg = logits.reshape(NUM_HEADS, SUM_POINTS, QPAD)
    m = jnp.max(lg, axis=1, keepdims=True)
    e = jnp.exp(lg - m)
    attn = e / jnp.sum(e, axis=1, keepdims=True)

    offs = offsT.reshape(NUM_HEADS, SUM_POINTS, 2, QPAD)
    ox = offs[:, :, 0, :]
    oy = offs[:, :, 1, :]

    refxy = refT_ref[0]
    refx = refxy[0:1, :]
    refy = refxy[1:2, :]
    refw = refxy[2:3, :]
    refh = refxy[3:4, :]

    p_iota = lax.broadcasted_iota(jnp.int32, (1, SUM_POINTS, 1), 1)
    nscale = jnp.where((p_iota >= 3) & (p_iota < 9),
                       jnp.float32(1.0 / 12.0), jnp.float32(1.0 / 6.0))

    locx = refx + ox * nscale * refw
    locy = refy + oy * nscale * refh
    x = locx * GW - 0.5
    y = locy * GH - 0.5
    x0 = jnp.floor(x)
    y0 = jnp.floor(y)
    fx = x - x0
    fy = y - y0
    x0i = x0.astype(jnp.int32)
    y0i = y0.astype(jnp.int32)

    qmask = (lax.broadcasted_iota(jnp.int32, (1, 1, QPAD), 2) < LEN_Q
             ).astype(jnp.float32)

    cell_list = []
    w_list = []
    for dx, dy, wgt in ((0, 0, (1 - fx) * (1 - fy)),
                        (0, 1, (1 - fx) * fy),
                        (1, 0, fx * (1 - fy)),
                        (1, 1, fx * fy)):
        ix = x0i + dx
        iy = y0i + dy
        valid = ((ix >= 0) & (ix < GW) & (iy >= 0) & (iy < GH)
                 ).astype(jnp.float32)
        cell = (jnp.clip(iy, 0, GH - 1) * GW + jnp.clip(ix, 0, GW - 1))
        cell_list.append(cell[:, :, None, :])
        w_list.append((wgt * valid * attn * qmask)[:, :, None, :])

    cells = jnp.concatenate(cell_list, axis=2).reshape(NUM_HEADS, NCORNER, QPAD)
    ws = jnp.concatenate(w_list, axis=2).reshape(NUM_HEADS, NCORNER, QPAD)
    cells_ref[0] = cells
    ws_ref[0] = ws


def _run_prep(qT, refT, WoffT, boff, WattnT, battn):
    grid = (BS,)
    return pl.pallas_call(
        _prep_body,
        grid=grid,
        in_specs=[
            pl.BlockSpec((1, EMBED_DIM, QPAD), lambda b: (b, 0, 0)),
            pl.BlockSpec((1, 4, QPAD), lambda b: (b, 0, 0)),
            pl.BlockSpec((SUM_POINTS * 16, EMBED_DIM), lambda b: (0, 0)),
            pl.BlockSpec((SUM_POINTS * 16, 1), lambda b: (0, 0)),
            pl.BlockSpec((SUM_POINTS * 8, EMBED_DIM), lambda b: (0, 0)),
            pl.BlockSpec((SUM_POINTS * 8, 1), lambda b: (0, 0)),
        ],
        out_specs=[
            pl.BlockSpec((1, NUM_HEADS, NCORNER, QPAD), lambda b: (b, 0, 0, 0)),
            pl.BlockSpec((1, NUM_HEADS, NCORNER, QPAD), lambda b: (b, 0, 0, 0)),
        ],
        out_shape=[
            jax.ShapeDtypeStruct((BS, NUM_HEADS, NCORNER, QPAD), jnp.int32),
            jax.ShapeDtypeStruct((BS, NUM_HEADS, NCORNER, QPAD), jnp.float32),
        ],
    )(qT, refT, WoffT, boff, WattnT, battn)


def _sc_body(value_hbm, cells_hbm, ws_hbm, out_hbm,
             slab0, slab1, cells_v, ws_v, out_v, sem0, sem1):
    cidx = lax.axis_index("c")
    sidx = lax.axis_index("s")
    wid = sidx * 2 + cidx

    slabs = (slab0, slab1)
    sems = (sem0, sem1)
    passes = [(lvl, half) for lvl in range(3) for half in range(2)]

    def start_dma(pair_i, pass_i):
        bh = wid * PAIRS_PER_WORKER + pair_i
        b = bh // NUM_HEADS
        h = bh % NUM_HEADS
        lvl, half = passes[pass_i]
        buf = pass_i % 2
        return pltpu.async_copy(
            value_hbm.at[lvl, b, h, pl.ds(half * 16, 16)], slabs[buf],
            sems[buf])

    descs = {(0, 0): start_dma(0, 0)}
    for i in range(PAIRS_PER_WORKER):
        bh = wid * PAIRS_PER_WORKER + i
        pltpu.sync_copy(cells_hbm.at[bh], cells_v)
        pltpu.sync_copy(ws_hbm.at[bh], ws_v)
        for p in range(6):
            lvl, half = passes[p]
            descs.pop((i, p)).wait()
            if p + 1 < 6:
                descs[(i, p + 1)] = start_dma(i, p + 1)
            elif i + 1 < PAIRS_PER_WORKER:
                descs[(i + 1, 0)] = start_dma(i + 1, 0)
            slab = slabs[p % 2]
            k0, k1 = KBOUNDS[lvl], KBOUNDS[lvl + 1]

            def qb_body(qb, carry, k0=k0, k1=k1, lvl=lvl, half=half, slab=slab):
                base = qb * 16

                def k_body(k, acc):
                    cellv = cells_v[k, pl.ds(base, 16)]
                    wv = ws_v[k, pl.ds(base, 16)]
                    out = []
                    for ch in range(16):
                        g = plsc.load_gather(
                            slab, [jnp.full((16,), ch, jnp.int32), cellv])
                        out.append(acc[ch] + g * wv)
                    return out

                if lvl == 0:
                    acc0 = [jnp.zeros((16,), jnp.float32)] * 16
                else:
                    acc0 = [out_v[half * 16 + ch, pl.ds(base, 16)]
                            for ch in range(16)]
                acc = lax.fori_loop(k0, k1, k_body, acc0)
                for ch in range(16):
                    out_v[half * 16 + ch, pl.ds(base, 16)] = acc[ch]
                return carry

            lax.fori_loop(0, QPAD // 16, qb_body, 0)
        pltpu.sync_copy(out_v, out_hbm.at[bh])


_sc_call_cache = []


def _get_sc_call():
    if not _sc_call_cache:
        mesh = plsc.VectorSubcoreMesh(core_axis_name="c", subcore_axis_name="s")
        _sc_call_cache.append(functools.partial(
            pl.kernel,
            out_type=jax.ShapeDtypeStruct((BH, HEAD_DIM, QPAD), jnp.float32),
            mesh=mesh,
            compiler_params=pltpu.CompilerParams(use_tc_tiling_on_sc=False,
                                                 needs_layout_passes=False),
            scratch_types=[
                pltpu.VMEM((16, GH * GW), jnp.float32),
                pltpu.VMEM((16, GH * GW), jnp.float32),
                pltpu.VMEM((NCORNER, QPAD), jnp.int32),
                pltpu.VMEM((NCORNER, QPAD), jnp.float32),
                pltpu.VMEM((HEAD_DIM, QPAD), jnp.float32),
                pltpu.SemaphoreType.DMA,
                pltpu.SemaphoreType.DMA,
            ],
        )(_sc_body))
    return _sc_call_cache[0]


def kernel(query, reference_points, value, value_spatial_shapes,
           W_off, b_off, W_attn, b_attn):
    del value_spatial_shapes
    qT = jnp.pad(jnp.swapaxes(query, 1, 2), ((0, 0), (0, 0), (0, QPAD - LEN_Q)))
    refT = jnp.pad(jnp.swapaxes(reference_points.reshape(BS, LEN_Q, 4), 1, 2),
                   ((0, 0), (0, 0), (0, QPAD - LEN_Q)))
    WoffT = jnp.swapaxes(W_off, 0, 1)
    WattnT = jnp.swapaxes(W_attn, 0, 1)
    boff = b_off.reshape(-1, 1)
    battn = b_attn.reshape(-1, 1)

    cells, ws = _run_prep(qT, refT, WoffT, boff, WattnT, battn)
    cells = cells.reshape(BH, NCORNER, QPAD)
    ws = ws.reshape(BH, NCORNER, QPAD)

    out = _get_sc_call()(value, cells, ws)

    out = out[:, :, :LEN_Q].reshape(BS, NUM_HEADS, HEAD_DIM, LEN_Q)
    return jnp.transpose(out, (0, 3, 1, 2)).reshape(BS, LEN_Q, EMBED_DIM)


# speedup vs baseline: 15.8101x; 1.4608x over previous
import functools

import jax
import jax.numpy as jnp
from jax import lax
from jax.experimental import pallas as pl
from jax.experimental.pallas import tpu as pltpu
from jax.experimental.pallas import tpu_sc as plsc

EMBED_DIM = 256
NUM_HEADS = 8
HEAD_DIM = 32
NUM_POINTS_LIST = (3, 6, 3)
SUM_POINTS = 12
NCORNER = 48
GH = GW = 40
BS = 16
LEN_Q = 300
QPAD = 384
QBLOCKS = 19
BH = BS * NUM_HEADS

KBOUNDS = (0, 12, 36, 48)

NUM_WORKERS = 32
PAIRS_PER_WORKER = BH // NUM_WORKERS


def _prep_body(qT_ref, refT_ref, WoffT_ref, boff_ref, WattnT_ref, battn_ref,
               cells_ref, ws_ref):
    qT = qT_ref[0]
    offsT = jnp.dot(WoffT_ref[...], qT, preferred_element_type=jnp.float32)
    offsT = offsT + boff_ref[...]
    logits = jnp.dot(WattnT_ref[...], qT, preferred_element_type=jnp.float32)
    logits = logits + battn_ref[...]

    lg = logits.reshape(NUM_HEADS, SUM_POINTS, QPAD)
    m = jnp.max(lg, axis=1, keepdims=True)
    e = jnp.exp(lg - m)
    attn = e / jnp.sum(e, axis=1, keepdims=True)

    offs = offsT.reshape(NUM_HEADS, SUM_POINTS, 2, QPAD)
    ox = offs[:, :, 0, :]
    oy = offs[:, :, 1, :]

    refxy = refT_ref[0]
    refx = refxy[0:1, :]
    refy = refxy[1:2, :]
    refw = refxy[2:3, :]
    refh = refxy[3:4, :]

    p_iota = lax.broadcasted_iota(jnp.int32, (1, SUM_POINTS, 1), 1)
    nscale = jnp.where((p_iota >= 3) & (p_iota < 9),
                       jnp.float32(1.0 / 12.0), jnp.float32(1.0 / 6.0))

    locx = refx + ox * nscale * refw
    locy = refy + oy * nscale * refh
    x = locx * GW - 0.5
    y = locy * GH - 0.5
    x0 = jnp.floor(x)
    y0 = jnp.floor(y)
    fx = x - x0
    fy = y - y0
    x0i = x0.astype(jnp.int32)
    y0i = y0.astype(jnp.int32)

    qmask = (lax.broadcasted_iota(jnp.int32, (1, 1, QPAD), 2) < LEN_Q
             ).astype(jnp.float32)

    cell_list = []
    w_list = []
    for dx, dy, wgt in ((0, 0, (1 - fx) * (1 - fy)),
                        (0, 1, (1 - fx) * fy),
                        (1, 0, fx * (1 - fy)),
                        (1, 1, fx * fy)):
        ix = x0i + dx
        iy = y0i + dy
        valid = ((ix >= 0) & (ix < GW) & (iy >= 0) & (iy < GH)
                 ).astype(jnp.float32)
        cell = (jnp.clip(iy, 0, GH - 1) * GW + jnp.clip(ix, 0, GW - 1))
        cell_list.append(cell[:, :, None, :])
        w_list.append((wgt * valid * attn * qmask)[:, :, None, :])

    cells = jnp.concatenate(cell_list, axis=2).reshape(NUM_HEADS, NCORNER, QPAD)
    ws = jnp.concatenate(w_list, axis=2).reshape(NUM_HEADS, NCORNER, QPAD)
    cells_ref[...] = cells
    ws_ref[...] = ws


def _run_prep(qT, refT, WoffT, boff, WattnT, battn):
    grid = (BS,)
    return pl.pallas_call(
        _prep_body,
        grid=grid,
        in_specs=[
            pl.BlockSpec((1, EMBED_DIM, QPAD), lambda b: (b, 0, 0)),
            pl.BlockSpec((1, 4, QPAD), lambda b: (b, 0, 0)),
            pl.BlockSpec((SUM_POINTS * 16, EMBED_DIM), lambda b: (0, 0)),
            pl.BlockSpec((SUM_POINTS * 16, 1), lambda b: (0, 0)),
            pl.BlockSpec((SUM_POINTS * 8, EMBED_DIM), lambda b: (0, 0)),
            pl.BlockSpec((SUM_POINTS * 8, 1), lambda b: (0, 0)),
        ],
        out_specs=[
            pl.BlockSpec((NUM_HEADS, NCORNER, QPAD), lambda b: (b, 0, 0)),
            pl.BlockSpec((NUM_HEADS, NCORNER, QPAD), lambda b: (b, 0, 0)),
        ],
        out_shape=[
            jax.ShapeDtypeStruct((BH, NCORNER, QPAD), jnp.int32),
            jax.ShapeDtypeStruct((BH, NCORNER, QPAD), jnp.float32),
        ],
    )(qT, refT, WoffT, boff, WattnT, battn)


def _sc_body(value_hbm, cells_hbm, ws_hbm, out_hbm,
             slab0, slab1, cells_v, ws_v, out_v, sem0, sem1):
    cidx = lax.axis_index("c")
    sidx = lax.axis_index("s")
    wid = sidx * 2 + cidx

    slabs = (slab0, slab1)
    sems = (sem0, sem1)
    passes = [(lvl, half) for lvl in range(3) for half in range(2)]

    def start_dma(pair_i, pass_i):
        bh = wid * PAIRS_PER_WORKER + pair_i
        b = bh // NUM_HEADS
        h = bh % NUM_HEADS
        lvl, half = passes[pass_i]
        buf = pass_i % 2
        return pltpu.async_copy(
            value_hbm.at[lvl, b, h, pl.ds(half * 16, 16)], slabs[buf],
            sems[buf])

    descs = {(0, 0): start_dma(0, 0)}
    for i in range(PAIRS_PER_WORKER):
        bh = wid * PAIRS_PER_WORKER + i
        pltpu.sync_copy(cells_hbm.at[bh], cells_v)
        pltpu.sync_copy(ws_hbm.at[bh], ws_v)
        for p in range(6):
            lvl, half = passes[p]
            descs.pop((i, p)).wait()
            if p + 1 < 6:
                descs[(i, p + 1)] = start_dma(i, p + 1)
            elif i + 1 < PAIRS_PER_WORKER:
                descs[(i + 1, 0)] = start_dma(i + 1, 0)
            slab = slabs[p % 2]
            k0, k1 = KBOUNDS[lvl], KBOUNDS[lvl + 1]

            def qb_body(qb, carry, k0=k0, k1=k1, lvl=lvl, half=half, slab=slab):
                base = qb * 16

                def k_body(k, acc):
                    cellv = cells_v[k, pl.ds(base, 16)]
                    wv = ws_v[k, pl.ds(base, 16)]
                    out = []
                    for ch in range(16):
                        g = plsc.load_gather(
                            slab, [jnp.full((16,), ch, jnp.int32), cellv])
                        out.append(acc[ch] + g * wv)
                    return out

                acc0 = [jnp.zeros((16,), jnp.float32)] * 16
                acc = lax.fori_loop(k0, k1, k_body, acc0)
                for ch in range(16):
                    if lvl == 0:
                        out_v[half * 16 + ch, pl.ds(base, 16)] = acc[ch]
                    else:
                        plsc.addupdate(
                            out_v.at[half * 16 + ch, pl.ds(base, 16)], acc[ch])
                return carry

            lax.fori_loop(0, QBLOCKS, qb_body, 0)
        pltpu.sync_copy(out_v, out_hbm.at[bh])


_sc_call_cache = []


def _get_sc_call():
    if not _sc_call_cache:
        mesh = plsc.VectorSubcoreMesh(core_axis_name="c", subcore_axis_name="s")
        _sc_call_cache.append(functools.partial(
            pl.kernel,
            out_type=jax.ShapeDtypeStruct((BH, HEAD_DIM, QPAD), jnp.float32),
            mesh=mesh,
            compiler_params=pltpu.CompilerParams(use_tc_tiling_on_sc=True,
                                                 needs_layout_passes=False),
            scratch_types=[
                pltpu.VMEM((16, GH * GW), jnp.float32),
                pltpu.VMEM((16, GH * GW), jnp.float32),
                pltpu.VMEM((NCORNER, QPAD), jnp.int32),
                pltpu.VMEM((NCORNER, QPAD), jnp.float32),
                pltpu.VMEM((HEAD_DIM, QPAD), jnp.float32),
                pltpu.SemaphoreType.DMA,
                pltpu.SemaphoreType.DMA,
            ],
        )(_sc_body))
    return _sc_call_cache[0]


def kernel(query, reference_points, value, value_spatial_shapes,
           W_off, b_off, W_attn, b_attn):
    del value_spatial_shapes
    qT = jnp.pad(jnp.swapaxes(query, 1, 2), ((0, 0), (0, 0), (0, QPAD - LEN_Q)))
    refT = jnp.pad(jnp.swapaxes(reference_points.reshape(BS, LEN_Q, 4), 1, 2),
                   ((0, 0), (0, 0), (0, QPAD - LEN_Q)))
    WoffT = jnp.swapaxes(W_off, 0, 1)
    WattnT = jnp.swapaxes(W_attn, 0, 1)
    boff = b_off.reshape(-1, 1)
    battn = b_attn.reshape(-1, 1)

    cells, ws = _run_prep(qT, refT, WoffT, boff, WattnT, battn)

    out = _get_sc_call()(value, cells, ws)

    out = out[:, :, :LEN_Q].reshape(BS, NUM_HEADS, HEAD_DIM, LEN_Q)
    return jnp.transpose(out, (0, 3, 1, 2)).reshape(BS, LEN_Q, EMBED_DIM)


# speedup vs baseline: 17.4129x; 1.1014x over previous
import functools

import jax
import jax.numpy as jnp
from jax import lax
from jax.experimental import pallas as pl
from jax.experimental.pallas import tpu as pltpu
from jax.experimental.pallas import tpu_sc as plsc

EMBED_DIM = 256
NUM_HEADS = 8
HEAD_DIM = 32
NUM_POINTS_LIST = (3, 6, 3)
SUM_POINTS = 12
NCORNER = 48
KCHUNKS = ((0, 16, 12), (16, 24, 24), (40, 16, 12))
NKROWS = 56
GH = GW = 40
BS = 16
LEN_Q = 300
QPAD = 384
QBLOCKS = 19
BH = BS * NUM_HEADS

KBOUNDS = (0, 12, 36, 48)

NUM_WORKERS = 32
PAIRS_PER_WORKER = BH // NUM_WORKERS


def _prep_body(q_ref, refT_ref, WT_ref, bias_ref, cells_ref, ws_ref):
    q = q_ref[0]
    proj = lax.dot_general(WT_ref[...], q, (((1,), (1,)), ((), ())),
                           preferred_element_type=jnp.float32)
    proj = proj + bias_ref[...]
    offsT = proj[0:192]
    logits = proj[192:288]

    lg = logits.reshape(NUM_HEADS, SUM_POINTS, QPAD)
    m = jnp.max(lg, axis=1, keepdims=True)
    e = jnp.exp(lg - m)
    attn = e / jnp.sum(e, axis=1, keepdims=True)

    offs = offsT.reshape(NUM_HEADS, SUM_POINTS, 2, QPAD)
    ox = offs[:, :, 0, :]
    oy = offs[:, :, 1, :]

    refxy = refT_ref[0]
    refx = refxy[0:1, :]
    refy = refxy[1:2, :]
    refw = refxy[2:3, :]
    refh = refxy[3:4, :]

    p_iota = lax.broadcasted_iota(jnp.int32, (1, SUM_POINTS, 1), 1)
    nscale = jnp.where((p_iota >= 3) & (p_iota < 9),
                       jnp.float32(1.0 / 12.0), jnp.float32(1.0 / 6.0))

    locx = refx + ox * nscale * refw
    locy = refy + oy * nscale * refh
    x = locx * GW - 0.5
    y = locy * GH - 0.5
    x0 = jnp.floor(x)
    y0 = jnp.floor(y)
    fx = x - x0
    fy = y - y0
    x0i = x0.astype(jnp.int32)
    y0i = y0.astype(jnp.int32)

    qmask = (lax.broadcasted_iota(jnp.int32, (1, 1, QPAD), 2) < LEN_Q
             ).astype(jnp.float32)

    x1i = x0i + 1
    y1i = y0i + 1
    vx0 = ((x0i >= 0) & (x0i < GW)).astype(jnp.float32)
    vx1 = ((x1i >= 0) & (x1i < GW)).astype(jnp.float32)
    vy0 = ((y0i >= 0) & (y0i < GH)).astype(jnp.float32)
    vy1 = ((y1i >= 0) & (y1i < GH)).astype(jnp.float32)
    cx0 = jnp.clip(x0i, 0, GW - 1)
    cx1 = jnp.clip(x1i, 0, GW - 1)
    ry0 = jnp.clip(y0i, 0, GH - 1) * GW
    ry1 = jnp.clip(y1i, 0, GH - 1) * GW
    aq = attn * qmask
    X0 = (1 - fx) * vx0
    X1 = fx * vx1
    Y0 = (1 - fy) * vy0 * aq
    Y1 = fy * vy1 * aq
    cell_cs = (ry0 + cx0, ry1 + cx0, ry0 + cx1, ry1 + cx1)
    w_cs = (X0 * Y0, X0 * Y1, X1 * Y0, X1 * Y1)

    zrow_i = jnp.zeros((NUM_HEADS, 4, QPAD), jnp.int32)
    zrow_f = jnp.zeros((NUM_HEADS, 4, QPAD), jnp.float32)

    def layout(arrs, zrow):
        parts = [a[:, 0:3] for a in arrs] + [zrow]
        parts += [a[:, 3:9] for a in arrs]
        parts += [a[:, 9:12] for a in arrs] + [zrow]
        return jnp.concatenate(parts, axis=1)

    cells_ref[...] = layout(cell_cs, zrow_i)
    ws_ref[...] = layout(w_cs, zrow_f)


def _run_prep(qpad, refT, WT, bias):
    grid = (BS,)
    return pl.pallas_call(
        _prep_body,
        grid=grid,
        in_specs=[
            pl.BlockSpec((1, QPAD, EMBED_DIM), lambda b: (b, 0, 0)),
            pl.BlockSpec((1, 4, QPAD), lambda b: (b, 0, 0)),
            pl.BlockSpec((SUM_POINTS * 24, EMBED_DIM), lambda b: (0, 0)),
            pl.BlockSpec((SUM_POINTS * 24, 1), lambda b: (0, 0)),
        ],
        out_specs=[
            pl.BlockSpec((NUM_HEADS, NKROWS, QPAD), lambda b: (b, 0, 0)),
            pl.BlockSpec((NUM_HEADS, NKROWS, QPAD), lambda b: (b, 0, 0)),
        ],
        out_shape=[
            jax.ShapeDtypeStruct((BH, NKROWS, QPAD), jnp.int32),
            jax.ShapeDtypeStruct((BH, NKROWS, QPAD), jnp.float32),
        ],
    )(qpad, refT, WT, bias)


def _sc_body(value_hbm, cells_hbm, ws_hbm, out_hbm,
             slab0, slab1, cells0, cells1, ws0, ws1, out_v,
             sem0, sem1, csem0, csem1, wsem0, wsem1):
    cidx = lax.axis_index("c")
    sidx = lax.axis_index("s")
    wid = sidx * 2 + cidx

    slabs = (slab0, slab1)
    sems = (sem0, sem1)
    cbufs = (cells0, cells1)
    csems = (csem0, csem1)
    wbufs = (ws0, ws1)
    wsems = (wsem0, wsem1)
    passes = [(lvl, half) for lvl in range(3) for half in range(2)]

    def start_slab(pair_i, pass_i):
        bh = wid * PAIRS_PER_WORKER + pair_i
        b = bh // NUM_HEADS
        h = bh % NUM_HEADS
        lvl, half = passes[pass_i]
        buf = pass_i % 2
        return pltpu.async_copy(
            value_hbm.at[lvl, b, h, pl.ds(half * 16, 16)], slabs[buf],
            sems[buf])

    def start_chunk(pair_i, lvl):
        bh = wid * PAIRS_PER_WORKER + pair_i
        row0, nrows, _ = KCHUNKS[lvl]
        c = (pair_i * 3 + lvl) % 2
        dc = pltpu.async_copy(cells_hbm.at[bh, pl.ds(row0, nrows)],
                              cbufs[c].at[pl.ds(0, nrows)], csems[c])
        dw = pltpu.async_copy(ws_hbm.at[bh, pl.ds(row0, nrows)],
                              wbufs[c].at[pl.ds(0, nrows)], wsems[c])
        return (dc, dw)

    descs = {(0, 0): start_slab(0, 0)}
    chunks = {(0, 0): start_chunk(0, 0)}
    for i in range(PAIRS_PER_WORKER):
        bh = wid * PAIRS_PER_WORKER + i
        for p in range(6):
            lvl, half = passes[p]
            descs.pop((i, p)).wait()
            if p + 1 < 6:
                descs[(i, p + 1)] = start_slab(i, p + 1)
            elif i + 1 < PAIRS_PER_WORKER:
                descs[(i + 1, 0)] = start_slab(i + 1, 0)
            slab = slabs[p % 2]
            cbuf_i = (i * 3 + lvl) % 2
            if half == 0:
                dc, dw = chunks.pop((i, lvl))
                dc.wait()
                dw.wait()
                if lvl + 1 < 3:
                    chunks[(i, lvl + 1)] = start_chunk(i, lvl + 1)
                elif i + 1 < PAIRS_PER_WORKER:
                    chunks[(i + 1, 0)] = start_chunk(i + 1, 0)
            cells_v = cbufs[cbuf_i]
            ws_v = wbufs[cbuf_i]

            def qb_body(qb, carry, lvl=lvl, half=half, slab=slab,
                        cells_v=cells_v, ws_v=ws_v, nk=KCHUNKS[lvl][2]):
                base = qb * 16

                def k_body(k, acc):
                    cellv = cells_v[k, pl.ds(base, 16)]
                    wv = ws_v[k, pl.ds(base, 16)]
                    out = []
                    for ch in range(16):
                        g = plsc.load_gather(
                            slab, [jnp.full((16,), ch, jnp.int32), cellv])
                        out.append(acc[ch] + g * wv)
                    return out

                acc0 = [jnp.zeros((16,), jnp.float32)] * 16
                acc = lax.fori_loop(0, nk, k_body, acc0)
                for ch in range(16):
                    if lvl == 0:
                        out_v[half * 16 + ch, pl.ds(base, 16)] = acc[ch]
                    else:
                        plsc.addupdate(
                            out_v.at[half * 16 + ch, pl.ds(base, 16)], acc[ch])
                return carry

            lax.fori_loop(0, QBLOCKS, qb_body, 0)
        pltpu.sync_copy(out_v, out_hbm.at[bh])


_sc_call_cache = []


def _get_sc_call():
    if not _sc_call_cache:
        mesh = plsc.VectorSubcoreMesh(core_axis_name="c", subcore_axis_name="s")
        _sc_call_cache.append(functools.partial(
            pl.kernel,
            out_type=jax.ShapeDtypeStruct((BH, HEAD_DIM, QPAD), jnp.float32),
            mesh=mesh,
            compiler_params=pltpu.CompilerParams(use_tc_tiling_on_sc=True,
                                                 needs_layout_passes=False),
            scratch_types=[
                pltpu.VMEM((16, GH * GW), jnp.float32),
                pltpu.VMEM((16, GH * GW), jnp.float32),
                pltpu.VMEM((24, QPAD), jnp.int32),
                pltpu.VMEM((24, QPAD), jnp.int32),
                pltpu.VMEM((24, QPAD), jnp.float32),
                pltpu.VMEM((24, QPAD), jnp.float32),
                pltpu.VMEM((HEAD_DIM, QPAD), jnp.float32),
                pltpu.SemaphoreType.DMA,
                pltpu.SemaphoreType.DMA,
                pltpu.SemaphoreType.DMA,
                pltpu.SemaphoreType.DMA,
                pltpu.SemaphoreType.DMA,
                pltpu.SemaphoreType.DMA,
            ],
        )(_sc_body))
    return _sc_call_cache[0]


def kernel(query, reference_points, value, value_spatial_shapes,
           W_off, b_off, W_attn, b_attn):
    del value_spatial_shapes
    qpad = jnp.pad(query, ((0, 0), (0, QPAD - LEN_Q), (0, 0)))
    refT = jnp.pad(jnp.swapaxes(reference_points.reshape(BS, LEN_Q, 4), 1, 2),
                   ((0, 0), (0, 0), (0, QPAD - LEN_Q)))
    WT = jnp.concatenate([jnp.swapaxes(W_off, 0, 1),
                          jnp.swapaxes(W_attn, 0, 1)], axis=0)
    bias = jnp.concatenate([b_off, b_attn]).reshape(-1, 1)

    cells, ws = _run_prep(qpad, refT, WT, bias)

    out = _get_sc_call()(value, cells, ws)

    out = out[:, :, :LEN_Q].reshape(BS, NUM_HEADS, HEAD_DIM, LEN_Q)
    return jnp.transpose(out, (0, 3, 1, 2)).reshape(BS, LEN_Q, EMBED_DIM)


# speedup vs baseline: 17.5076x; 1.0054x over previous
import functools

import jax
import jax.numpy as jnp
from jax import lax
from jax.experimental import pallas as pl
from jax.experimental.pallas import tpu as pltpu
from jax.experimental.pallas import tpu_sc as plsc

EMBED_DIM = 256
NUM_HEADS = 8
HEAD_DIM = 32
NUM_POINTS_LIST = (3, 6, 3)
SUM_POINTS = 12
NCORNER = 48
KCHUNKS = ((0, 16, 12), (16, 24, 24), (40, 16, 12))
NKROWS = 56
GH = GW = 40
BS = 16
LEN_Q = 300
QPAD = 384
QBLOCKS = 19
BH = BS * NUM_HEADS

KBOUNDS = (0, 12, 36, 48)

NUM_WORKERS = 32
PAIRS_PER_WORKER = BH // NUM_WORKERS


def _prep_body(q_ref, refT_ref, WT_ref, bias_ref, cells_ref, ws_ref):
    q = q_ref[0]
    proj = lax.dot_general(WT_ref[...], q, (((1,), (1,)), ((), ())),
                           preferred_element_type=jnp.float32)
    proj = proj + bias_ref[...]
    offsT = proj[0:192]
    logits = proj[192:288]

    lg = logits.reshape(NUM_HEADS, SUM_POINTS, QPAD)
    m = jnp.max(lg, axis=1, keepdims=True)
    e = jnp.exp(lg - m)
    attn = e / jnp.sum(e, axis=1, keepdims=True)

    offs = offsT.reshape(NUM_HEADS, SUM_POINTS, 2, QPAD)
    ox = offs[:, :, 0, :]
    oy = offs[:, :, 1, :]

    refxy = refT_ref[0]
    refx = refxy[0:1, :]
    refy = refxy[1:2, :]
    refw = refxy[2:3, :]
    refh = refxy[3:4, :]

    p_iota = lax.broadcasted_iota(jnp.int32, (1, SUM_POINTS, 1), 1)
    nscale = jnp.where((p_iota >= 3) & (p_iota < 9),
                       jnp.float32(1.0 / 12.0), jnp.float32(1.0 / 6.0))

    locx = refx + ox * nscale * refw
    locy = refy + oy * nscale * refh
    x = locx * GW - 0.5
    y = locy * GH - 0.5
    x0 = jnp.floor(x)
    y0 = jnp.floor(y)
    fx = x - x0
    fy = y - y0
    x0i = x0.astype(jnp.int32)
    y0i = y0.astype(jnp.int32)

    qmask = (lax.broadcasted_iota(jnp.int32, (1, 1, QPAD), 2) < LEN_Q
             ).astype(jnp.float32)

    x1i = x0i + 1
    y1i = y0i + 1
    vx0 = ((x0i >= 0) & (x0i < GW)).astype(jnp.float32)
    vx1 = ((x1i >= 0) & (x1i < GW)).astype(jnp.float32)
    vy0 = ((y0i >= 0) & (y0i < GH)).astype(jnp.float32)
    vy1 = ((y1i >= 0) & (y1i < GH)).astype(jnp.float32)
    cx0 = jnp.clip(x0i, 0, GW - 1)
    cx1 = jnp.clip(x1i, 0, GW - 1)
    ry0 = jnp.clip(y0i, 0, GH - 1) * GW
    ry1 = jnp.clip(y1i, 0, GH - 1) * GW
    aq = attn * qmask
    X0 = (1 - fx) * vx0
    X1 = fx * vx1
    Y0 = (1 - fy) * vy0 * aq
    Y1 = fy * vy1 * aq
    cell_cs = (ry0 + cx0, ry1 + cx0, ry0 + cx1, ry1 + cx1)
    w_cs = (X0 * Y0, X0 * Y1, X1 * Y0, X1 * Y1)

    zrow_i = jnp.zeros((NUM_HEADS, 4, QPAD), jnp.int32)
    zrow_f = jnp.zeros((NUM_HEADS, 4, QPAD), jnp.float32)

    def layout(arrs, zrow):
        parts = [a[:, 0:3] for a in arrs] + [zrow]
        parts += [a[:, 3:9] for a in arrs]
        parts += [a[:, 9:12] for a in arrs] + [zrow]
        return jnp.concatenate(parts, axis=1)

    cells_ref[...] = layout(cell_cs, zrow_i)
    ws_ref[...] = layout(w_cs, zrow_f)


def _run_prep(qpad, refT, WT, bias):
    grid = (BS,)
    return pl.pallas_call(
        _prep_body,
        grid=grid,
        in_specs=[
            pl.BlockSpec((1, QPAD, EMBED_DIM), lambda b: (b, 0, 0)),
            pl.BlockSpec((1, 4, QPAD), lambda b: (b, 0, 0)),
            pl.BlockSpec((SUM_POINTS * 24, EMBED_DIM), lambda b: (0, 0)),
            pl.BlockSpec((SUM_POINTS * 24, 1), lambda b: (0, 0)),
        ],
        out_specs=[
            pl.BlockSpec((NUM_HEADS, NKROWS, QPAD), lambda b: (b, 0, 0)),
            pl.BlockSpec((NUM_HEADS, NKROWS, QPAD), lambda b: (b, 0, 0)),
        ],
        out_shape=[
            jax.ShapeDtypeStruct((BH, NKROWS, QPAD), jnp.int32),
            jax.ShapeDtypeStruct((BH, NKROWS, QPAD), jnp.float32),
        ],
    )(qpad, refT, WT, bias)


def _sc_body(value_hbm, cells_hbm, ws_hbm, out_hbm,
             slab0, slab1, cells0, cells1, ws0, ws1, out_v,
             sem0, sem1, csem0, csem1, wsem0, wsem1):
    cidx = lax.axis_index("c")
    sidx = lax.axis_index("s")
    wid = sidx * 2 + cidx

    slabs = (slab0, slab1)
    sems = (sem0, sem1)
    cbufs = (cells0, cells1)
    csems = (csem0, csem1)
    wbufs = (ws0, ws1)
    wsems = (wsem0, wsem1)
    passes = [(lvl, half) for lvl in range(3) for half in range(2)]

    def start_slab(pair_i, pass_i):
        bh = wid * PAIRS_PER_WORKER + pair_i
        b = bh // NUM_HEADS
        h = bh % NUM_HEADS
        lvl, half = passes[pass_i]
        buf = pass_i % 2
        return pltpu.async_copy(
            value_hbm.at[lvl, b, h, pl.ds(half * 16, 16)], slabs[buf],
            sems[buf])

    def start_chunk(pair_i, lvl):
        bh = wid * PAIRS_PER_WORKER + pair_i
        row0, nrows, _ = KCHUNKS[lvl]
        c = (pair_i * 3 + lvl) % 2
        dc = pltpu.async_copy(cells_hbm.at[bh, pl.ds(row0, nrows)],
                              cbufs[c].at[pl.ds(0, nrows)], csems[c])
        dw = pltpu.async_copy(ws_hbm.at[bh, pl.ds(row0, nrows)],
                              wbufs[c].at[pl.ds(0, nrows)], wsems[c])
        return (dc, dw)

    descs = {(0, 0): start_slab(0, 0)}
    chunks = {(0, 0): start_chunk(0, 0)}
    for i in range(PAIRS_PER_WORKER):
        bh = wid * PAIRS_PER_WORKER + i
        for p in range(6):
            lvl, half = passes[p]
            descs.pop((i, p)).wait()
            if p + 1 < 6:
                descs[(i, p + 1)] = start_slab(i, p + 1)
            elif i + 1 < PAIRS_PER_WORKER:
                descs[(i + 1, 0)] = start_slab(i + 1, 0)
            slab = slabs[p % 2]
            cbuf_i = (i * 3 + lvl) % 2
            if half == 0:
                dc, dw = chunks.pop((i, lvl))
                dc.wait()
                dw.wait()
                if lvl + 1 < 3:
                    chunks[(i, lvl + 1)] = start_chunk(i, lvl + 1)
                elif i + 1 < PAIRS_PER_WORKER:
                    chunks[(i + 1, 0)] = start_chunk(i + 1, 0)
            cells_v = cbufs[cbuf_i]
            ws_v = wbufs[cbuf_i]

            def qb_body(qb, carry, lvl=lvl, half=half, slab=slab,
                        cells_v=cells_v, ws_v=ws_v, nk=KCHUNKS[lvl][2]):
                base = qb * 16

                def k_body(k, acc):
                    cellv = cells_v[k, pl.ds(base, 16)]
                    wv = ws_v[k, pl.ds(base, 16)]
                    out = []
                    for ch in range(16):
                        g = plsc.load_gather(
                            slab, [jnp.full((16,), ch, jnp.int32), cellv])
                        out.append(acc[ch] + g * wv)
                    return out

                acc0 = [jnp.zeros((16,), jnp.float32)] * 16
                acc = plsc.parallel_loop(0, nk, unroll=2, carry=acc0)(
                    lambda k, acc: k_body(k, acc))
                for ch in range(16):
                    if lvl == 0:
                        out_v[half * 16 + ch, pl.ds(base, 16)] = acc[ch]
                    else:
                        plsc.addupdate(
                            out_v.at[half * 16 + ch, pl.ds(base, 16)], acc[ch])
                return carry

            lax.fori_loop(0, QBLOCKS, qb_body, 0)
        pltpu.sync_copy(out_v, out_hbm.at[bh])


_sc_call_cache = []


def _get_sc_call():
    if not _sc_call_cache:
        mesh = plsc.VectorSubcoreMesh(core_axis_name="c", subcore_axis_name="s")
        _sc_call_cache.append(functools.partial(
            pl.kernel,
            out_type=jax.ShapeDtypeStruct((BH, HEAD_DIM, QPAD), jnp.float32),
            mesh=mesh,
            compiler_params=pltpu.CompilerParams(use_tc_tiling_on_sc=True,
                                                 needs_layout_passes=False),
            scratch_types=[
                pltpu.VMEM((16, GH * GW), jnp.float32),
                pltpu.VMEM((16, GH * GW), jnp.float32),
                pltpu.VMEM((24, QPAD), jnp.int32),
                pltpu.VMEM((24, QPAD), jnp.int32),
                pltpu.VMEM((24, QPAD), jnp.float32),
                pltpu.VMEM((24, QPAD), jnp.float32),
                pltpu.VMEM((HEAD_DIM, QPAD), jnp.float32),
                pltpu.SemaphoreType.DMA,
                pltpu.SemaphoreType.DMA,
                pltpu.SemaphoreType.DMA,
                pltpu.SemaphoreType.DMA,
                pltpu.SemaphoreType.DMA,
                pltpu.SemaphoreType.DMA,
            ],
        )(_sc_body))
    return _sc_call_cache[0]


def kernel(query, reference_points, value, value_spatial_shapes,
           W_off, b_off, W_attn, b_attn):
    del value_spatial_shapes
    qpad = jnp.pad(query, ((0, 0), (0, QPAD - LEN_Q), (0, 0)))
    refT = jnp.pad(jnp.swapaxes(reference_points.reshape(BS, LEN_Q, 4), 1, 2),
                   ((0, 0), (0, 0), (0, QPAD - LEN_Q)))
    WT = jnp.concatenate([jnp.swapaxes(W_off, 0, 1),
                          jnp.swapaxes(W_attn, 0, 1)], axis=0)
    bias = jnp.concatenate([b_off, b_attn]).reshape(-1, 1)

    cells, ws = _run_prep(qpad, refT, WT, bias)

    out = _get_sc_call()(value, cells, ws)

    out = out[:, :, :LEN_Q].reshape(BS, NUM_HEADS, HEAD_DIM, LEN_Q)
    return jnp.transpose(out, (0, 3, 1, 2)).reshape(BS, LEN_Q, EMBED_DIM)


# speedup vs baseline: 18.2678x; 1.0434x over previous
import functools

import jax
import jax.numpy as jnp
from jax import lax
from jax.experimental import pallas as pl
from jax.experimental.pallas import tpu as pltpu
from jax.experimental.pallas import tpu_sc as plsc

EMBED_DIM = 256
NUM_HEADS = 8
HEAD_DIM = 32
NUM_POINTS_LIST = (3, 6, 3)
SUM_POINTS = 12
NCORNER = 48
KCHUNKS = ((0, 16, 12), (16, 24, 24), (40, 16, 12))
NKROWS = 56
GH = GW = 40
BS = 16
LEN_Q = 300
QPAD = 384
QBLOCKS = 19
BH = BS * NUM_HEADS

KBOUNDS = (0, 12, 36, 48)

NUM_WORKERS = 32
PAIRS_PER_WORKER = BH // NUM_WORKERS


def _prep_body(q_ref, refT_ref, WT_ref, bias_ref, cells_ref, ws_ref):
    q = q_ref[0]
    proj = lax.dot_general(WT_ref[...], q, (((1,), (1,)), ((), ())),
                           preferred_element_type=jnp.float32)
    proj = proj + bias_ref[...]
    offsT = proj[0:192]
    logits = proj[192:288]

    lg = logits.reshape(NUM_HEADS, SUM_POINTS, QPAD)
    m = jnp.max(lg, axis=1, keepdims=True)
    e = jnp.exp(lg - m)
    attn = e / jnp.sum(e, axis=1, keepdims=True)

    offs = offsT.reshape(NUM_HEADS, SUM_POINTS, 2, QPAD)
    ox = offs[:, :, 0, :]
    oy = offs[:, :, 1, :]

    refxy = refT_ref[0]
    refx = refxy[0:1, :]
    refy = refxy[1:2, :]
    refw = refxy[2:3, :]
    refh = refxy[3:4, :]

    p_iota = lax.broadcasted_iota(jnp.int32, (1, SUM_POINTS, 1), 1)
    nscale = jnp.where((p_iota >= 3) & (p_iota < 9),
                       jnp.float32(1.0 / 12.0), jnp.float32(1.0 / 6.0))

    locx = refx + ox * nscale * refw
    locy = refy + oy * nscale * refh
    x = locx * GW - 0.5
    y = locy * GH - 0.5
    x0 = jnp.floor(x)
    y0 = jnp.floor(y)
    fx = x - x0
    fy = y - y0
    x0i = x0.astype(jnp.int32)
    y0i = y0.astype(jnp.int32)

    qmask = (lax.broadcasted_iota(jnp.int32, (1, 1, QPAD), 2) < LEN_Q
             ).astype(jnp.float32)

    x1i = x0i + 1
    y1i = y0i + 1
    vx0 = ((x0i >= 0) & (x0i < GW)).astype(jnp.float32)
    vx1 = ((x1i >= 0) & (x1i < GW)).astype(jnp.float32)
    vy0 = ((y0i >= 0) & (y0i < GH)).astype(jnp.float32)
    vy1 = ((y1i >= 0) & (y1i < GH)).astype(jnp.float32)
    cx0 = jnp.clip(x0i, 0, GW - 1)
    cx1 = jnp.clip(x1i, 0, GW - 1)
    ry0 = jnp.clip(y0i, 0, GH - 1) * GW
    ry1 = jnp.clip(y1i, 0, GH - 1) * GW
    aq = attn * qmask
    X0 = (1 - fx) * vx0
    X1 = fx * vx1
    Y0 = (1 - fy) * vy0 * aq
    Y1 = fy * vy1 * aq
    cell_cs = (ry0 + cx0, ry1 + cx0, ry0 + cx1, ry1 + cx1)
    w_cs = (X0 * Y0, X0 * Y1, X1 * Y0, X1 * Y1)

    zrow_i = jnp.zeros((NUM_HEADS, 4, QPAD), jnp.int32)
    zrow_f = jnp.zeros((NUM_HEADS, 4, QPAD), jnp.float32)

    def layout(arrs, zrow):
        parts = [a[:, 0:3] for a in arrs] + [zrow]
        parts += [a[:, 3:9] for a in arrs]
        parts += [a[:, 9:12] for a in arrs] + [zrow]
        return jnp.concatenate(parts, axis=1)

    cells_ref[...] = layout(cell_cs, zrow_i)
    ws_ref[...] = layout(w_cs, zrow_f)


def _run_prep(qpad, refT, WT, bias):
    grid = (BS,)
    return pl.pallas_call(
        _prep_body,
        grid=grid,
        in_specs=[
            pl.BlockSpec((1, QPAD, EMBED_DIM), lambda b: (b, 0, 0)),
            pl.BlockSpec((1, 4, QPAD), lambda b: (b, 0, 0)),
            pl.BlockSpec((SUM_POINTS * 24, EMBED_DIM), lambda b: (0, 0)),
            pl.BlockSpec((SUM_POINTS * 24, 1), lambda b: (0, 0)),
        ],
        out_specs=[
            pl.BlockSpec((NUM_HEADS, NKROWS, QPAD), lambda b: (b, 0, 0)),
            pl.BlockSpec((NUM_HEADS, NKROWS, QPAD), lambda b: (b, 0, 0)),
        ],
        out_shape=[
            jax.ShapeDtypeStruct((BH, NKROWS, QPAD), jnp.int32),
            jax.ShapeDtypeStruct((BH, NKROWS, QPAD), jnp.float32),
        ],
    )(qpad, refT, WT, bias)


def _sc_body(value_hbm, cells_hbm, ws_hbm, out_hbm,
             slab0, slab1, cells0, cells1, ws0, ws1, out_v,
             sem0, sem1, csem0, csem1, wsem0, wsem1):
    cidx = lax.axis_index("c")
    sidx = lax.axis_index("s")
    wid = sidx * 2 + cidx

    slabs = (slab0, slab1)
    sems = (sem0, sem1)
    cbufs = (cells0, cells1)
    csems = (csem0, csem1)
    wbufs = (ws0, ws1)
    wsems = (wsem0, wsem1)
    passes = [(lvl, half) for lvl in range(3) for half in range(2)]

    def start_slab(pair_i, pass_i):
        bh = wid * PAIRS_PER_WORKER + pair_i
        b = bh // NUM_HEADS
        h = bh % NUM_HEADS
        lvl, half = passes[pass_i]
        buf = pass_i % 2
        return pltpu.async_copy(
            value_hbm.at[lvl, b, h, pl.ds(half * 16, 16)], slabs[buf],
            sems[buf])

    def start_chunk(pair_i, lvl):
        bh = wid * PAIRS_PER_WORKER + pair_i
        row0, nrows, _ = KCHUNKS[lvl]
        c = (pair_i * 3 + lvl) % 2
        dc = pltpu.async_copy(cells_hbm.at[bh, pl.ds(row0, nrows)],
                              cbufs[c].at[pl.ds(0, nrows)], csems[c])
        dw = pltpu.async_copy(ws_hbm.at[bh, pl.ds(row0, nrows)],
                              wbufs[c].at[pl.ds(0, nrows)], wsems[c])
        return (dc, dw)

    descs = {(0, 0): start_slab(0, 0)}
    chunks = {(0, 0): start_chunk(0, 0)}
    for i in range(PAIRS_PER_WORKER):
        bh = wid * PAIRS_PER_WORKER + i
        for p in range(6):
            lvl, half = passes[p]
            descs.pop((i, p)).wait()
            if p + 1 < 6:
                descs[(i, p + 1)] = start_slab(i, p + 1)
            elif i + 1 < PAIRS_PER_WORKER:
                descs[(i + 1, 0)] = start_slab(i + 1, 0)
            slab = slabs[p % 2]
            cbuf_i = (i * 3 + lvl) % 2
            if half == 0:
                dc, dw = chunks.pop((i, lvl))
                dc.wait()
                dw.wait()
                if lvl + 1 < 3:
                    chunks[(i, lvl + 1)] = start_chunk(i, lvl + 1)
                elif i + 1 < PAIRS_PER_WORKER:
                    chunks[(i + 1, 0)] = start_chunk(i + 1, 0)
            cells_v = cbufs[cbuf_i]
            ws_v = wbufs[cbuf_i]

            def qb_body(qb, carry, lvl=lvl, half=half, slab=slab,
                        cells_v=cells_v, ws_v=ws_v, nk=KCHUNKS[lvl][2]):
                base = qb * 16

                def k_body(k, acc):
                    cellv = cells_v[k, pl.ds(base, 16)]
                    wv = ws_v[k, pl.ds(base, 16)]
                    out = []
                    for ch in range(16):
                        g = plsc.load_gather(
                            slab, [jnp.full((16,), ch, jnp.int32), cellv])
                        out.append(acc[ch] + g * wv)
                    return out

                acc0 = [jnp.zeros((16,), jnp.float32)] * 16
                acc = plsc.parallel_loop(0, nk, unroll=4, carry=acc0)(
                    lambda k, acc: k_body(k, acc))
                for ch in range(16):
                    if lvl == 0:
                        out_v[half * 16 + ch, pl.ds(base, 16)] = acc[ch]
                    else:
                        plsc.addupdate(
                            out_v.at[half * 16 + ch, pl.ds(base, 16)], acc[ch])
                return carry

            lax.fori_loop(0, QBLOCKS, qb_body, 0)
        pltpu.sync_copy(out_v, out_hbm.at[bh])


_sc_call_cache = []


def _get_sc_call():
    if not _sc_call_cache:
        mesh = plsc.VectorSubcoreMesh(core_axis_name="c", subcore_axis_name="s")
        _sc_call_cache.append(functools.partial(
            pl.kernel,
            out_type=jax.ShapeDtypeStruct((BH, HEAD_DIM, QPAD), jnp.float32),
            mesh=mesh,
            compiler_params=pltpu.CompilerParams(use_tc_tiling_on_sc=True,
                                                 needs_layout_passes=False),
            scratch_types=[
                pltpu.VMEM((16, GH * GW), jnp.float32),
                pltpu.VMEM((16, GH * GW), jnp.float32),
                pltpu.VMEM((24, QPAD), jnp.int32),
                pltpu.VMEM((24, QPAD), jnp.int32),
                pltpu.VMEM((24, QPAD), jnp.float32),
                pltpu.VMEM((24, QPAD), jnp.float32),
                pltpu.VMEM((HEAD_DIM, QPAD), jnp.float32),
                pltpu.SemaphoreType.DMA,
                pltpu.SemaphoreType.DMA,
                pltpu.SemaphoreType.DMA,
                pltpu.SemaphoreType.DMA,
                pltpu.SemaphoreType.DMA,
                pltpu.SemaphoreType.DMA,
            ],
        )(_sc_body))
    return _sc_call_cache[0]


def kernel(query, reference_points, value, value_spatial_shapes,
           W_off, b_off, W_attn, b_attn):
    del value_spatial_shapes
    qpad = jnp.pad(query, ((0, 0), (0, QPAD - LEN_Q), (0, 0)))
    refT = jnp.pad(jnp.swapaxes(reference_points.reshape(BS, LEN_Q, 4), 1, 2),
                   ((0, 0), (0, 0), (0, QPAD - LEN_Q)))
    WT = jnp.concatenate([jnp.swapaxes(W_off, 0, 1),
                          jnp.swapaxes(W_attn, 0, 1)], axis=0)
    bias = jnp.concatenate([b_off, b_attn]).reshape(-1, 1)

    cells, ws = _run_prep(qpad, refT, WT, bias)

    out = _get_sc_call()(value, cells, ws)

    out = out[:, :, :LEN_Q].reshape(BS, NUM_HEADS, HEAD_DIM, LEN_Q)
    return jnp.transpose(out, (0, 3, 1, 2)).reshape(BS, LEN_Q, EMBED_DIM)
